# baseline (device time: 88245 ns/iter reference)
import jax
import jax.numpy as jnp
from jax import lax
from jax.experimental import pallas as pl
from jax.experimental.pallas import tpu as pltpu

N_Y = 4
N_Z = 4
N_X = 2
M = 4096
R = M // (N_Z * N_X)
N_COLS = 4096
CHUNK = N_COLS // N_Y
P = 4
W = CHUNK // P
OFF = 1
N_EV = 5


def kernel(x):
    x2d = x[0]

    def body(
        x_hbm,
        out_ref,
        ybuf_send,
        ybuf_recv,
        stage,
        copy_sems,
        y_send,
        y_recv,
        z_send,
        z_recv,
        x_send,
        x_recv,
    ):
        my_x = lax.axis_index("x")
        my_y = lax.axis_index("y")
        my_z = lax.axis_index("z")
        other_x = lax.rem(my_x + 1, N_X)

        barrier = pltpu.get_barrier_semaphore()
        for j in range(1, N_Y):
            pl.semaphore_signal(
                barrier, inc=1,
                device_id=(my_x, lax.rem(my_y + j, N_Y), my_z),
                device_id_type=pl.DeviceIdType.MESH,
            )
        for j in range(1, N_Z):
            pl.semaphore_signal(
                barrier, inc=1,
                device_id=(my_x, my_y, lax.rem(my_z + j, N_Z)),
                device_id_type=pl.DeviceIdType.MESH,
            )
        pl.semaphore_signal(
            barrier, inc=1,
            device_id=(other_x, my_y, my_z),
            device_id_type=pl.DeviceIdType.MESH,
        )
        pl.semaphore_wait(barrier, N_Y - 1 + N_Z - 1 + 1)

        my_b = N_X * my_z + my_x
        row0 = my_b * R

        def stage_dma(chunk_idx, h, j):
            return pltpu.make_async_copy(
                x_hbm.at[
                    pl.ds(row0, R),
                    pl.ds(chunk_idx * CHUNK + h * W, W),
                ],
                stage.at[h, j],
                copy_sems.at[h, j],
            )

        def y_rdma(h, j):
            return pltpu.make_async_remote_copy(
                src_ref=ybuf_send.at[h, j],
                dst_ref=ybuf_recv.at[h, j],
                send_sem=y_send.at[h, j],
                recv_sem=y_recv.at[h, j],
                device_id=(my_x, lax.rem(my_y + j + 1, N_Y), my_z),
                device_id_type=pl.DeviceIdType.MESH,
            )

        def blk_rows(t, xi):
            zp = lax.rem(my_z + N_Z - t, N_Z)
            return (N_X * zp + xi) * R

        def z_send_rdma(h, j):
            sl = (pl.ds(row0, R), pl.ds(h * W, W))
            return pltpu.make_async_remote_copy(
                src_ref=out_ref.at[sl],
                dst_ref=out_ref.at[sl],
                send_sem=z_send.at[h, j - 1],
                recv_sem=z_recv.at[h, j - 1],
                device_id=(my_x, my_y, lax.rem(my_z + j, N_Z)),
                device_id_type=pl.DeviceIdType.MESH,
            )

        def z_recv_desc(h, j):
            sl = (pl.ds(blk_rows(j, my_x), R), pl.ds(h * W, W))
            return pltpu.make_async_remote_copy(
                src_ref=out_ref.at[sl],
                dst_ref=out_ref.at[sl],
                send_sem=z_send.at[h, j - 1],
                recv_sem=z_recv.at[h, j - 1],
                device_id=(my_x, my_y, lax.rem(my_z + j, N_Z)),
                device_id_type=pl.DeviceIdType.MESH,
            )

        def x_pair(h, t):
            sl_mine = (pl.ds(blk_rows(t, my_x), R), pl.ds(h * W, W))
            sl_theirs = (pl.ds(blk_rows(t, other_x), R), pl.ds(h * W, W))
            send = pltpu.make_async_remote_copy(
                src_ref=out_ref.at[sl_mine],
                dst_ref=out_ref.at[sl_mine],
                send_sem=x_send.at[h, t],
                recv_sem=x_recv.at[h, t],
                device_id=(other_x, my_y, my_z),
                device_id_type=pl.DeviceIdType.MESH,
            )
            recv = pltpu.make_async_remote_copy(
                src_ref=out_ref.at[sl_theirs],
                dst_ref=out_ref.at[sl_theirs],
                send_sem=x_send.at[h, t],
                recv_sem=x_recv.at[h, t],
                device_id=(other_x, my_y, my_z),
                device_id_type=pl.DeviceIdType.MESH,
            )
            return send, recv

        y_d = [[y_rdma(h, j) for j in range(N_Y - 1)] for h in range(P)]
        zs_d = [[z_send_rdma(h, j) for j in range(1, N_Z)] for h in range(P)]
        zr_d = [[z_recv_desc(h, j) for j in range(1, N_Z)] for h in range(P)]
        x_pairs = [[x_pair(h, t) for t in range(N_Z)] for h in range(P)]

        st_dmas = []
        for h in range(P):
            per_h = []
            for j in range(N_Y - 1):
                d = stage_dma(lax.rem(my_y + j + 1, N_Y), h, j)
                d.start()
                per_h.append(d)
            d = stage_dma(my_y, h, 3)
            d.start()
            per_h.append(d)
            st_dmas.append(per_h)

        def event(h, e):
            if e == 0:
                for j in range(N_Y - 1):
                    st_dmas[h][j].wait()
                    ybuf_send[h, j] = stage[h, j].astype(jnp.bfloat16)
                    y_d[h][j].start()
            elif e == 1:
                for j in range(N_Y - 1):
                    y_d[h][j].wait()
                st_dmas[h][3].wait()
                out_ref[pl.ds(row0, R), pl.ds(h * W, W)] = (
                    stage[h, 3].astype(jnp.bfloat16)
                    + ybuf_recv[h, 0]
                    + ybuf_recv[h, 1]
                    + ybuf_recv[h, 2]
                )
                x_pairs[h][0][0].start()
                for j in range(N_Z - 1):
                    zs_d[h][j].start()
            else:
                j = e - 2
                zr_d[h][j].wait_recv()
                x_pairs[h][j + 1][0].start()

        for k in range(N_EV + (P - 1) * OFF):
            for h in range(P):
                e = k - h * OFF
                if 0 <= e < N_EV:
                    event(h, e)

        for h in range(P):
            for j in range(N_Z - 1):
                zs_d[h][j].wait_send()
            for t in range(N_Z):
                x_pairs[h][t][0].wait_send()
                x_pairs[h][t][1].wait_recv()

    return pl.pallas_call(
        body,
        out_shape=jax.ShapeDtypeStruct((M, CHUNK), jnp.bfloat16),
        in_specs=[pl.BlockSpec(memory_space=pl.ANY)],
        out_specs=pl.BlockSpec(memory_space=pltpu.VMEM),
        scratch_shapes=[
            pltpu.VMEM((P, N_Y - 1, R, W), jnp.bfloat16),
            pltpu.VMEM((P, N_Y - 1, R, W), jnp.bfloat16),
            pltpu.VMEM((P, N_Y, R, W), jnp.float32),
            pltpu.SemaphoreType.DMA((P, N_Y)),
            pltpu.SemaphoreType.DMA((P, N_Y - 1)),
            pltpu.SemaphoreType.DMA((P, N_Y - 1)),
            pltpu.SemaphoreType.DMA((P, N_Z - 1)),
            pltpu.SemaphoreType.DMA((P, N_Z - 1)),
            pltpu.SemaphoreType.DMA((P, N_Z)),
            pltpu.SemaphoreType.DMA((P, N_Z)),
        ],
        compiler_params=pltpu.CompilerParams(
            collective_id=0,
            vmem_limit_bytes=48 * 1024 * 1024,
        ),
    )(x2d)


# device time: 86726 ns/iter; 1.0175x vs baseline; 1.0175x over previous
import jax
import jax.numpy as jnp
from jax import lax
from jax.experimental import pallas as pl
from jax.experimental.pallas import tpu as pltpu

N_Y = 4
N_Z = 4
N_X = 2
M = 4096
R = M // (N_Z * N_X)
N_COLS = 4096
CHUNK = N_COLS // N_Y
P = 4
W = CHUNK // P
OFF = 1
N_EV = 7


def kernel(x):
    x2d = x[0]

    def body(
        x_hbm,
        out_ref,
        rs_comm,
        stage,
        copy_sems,
        rs_send,
        rs_recv,
        ag_send,
        ag_recv,
        x_send,
        x_recv,
    ):
        my_x = lax.axis_index("x")
        my_y = lax.axis_index("y")
        my_z = lax.axis_index("z")
        other_x = lax.rem(my_x + 1, N_X)
        y_right = lax.rem(my_y + 1, N_Y)
        y_left = lax.rem(my_y + N_Y - 1, N_Y)
        z_right = lax.rem(my_z + 1, N_Z)
        z_left = lax.rem(my_z + N_Z - 1, N_Z)

        barrier = pltpu.get_barrier_semaphore()
        for nbr_dev in (
            (my_x, y_left, my_z),
            (my_x, y_right, my_z),
            (my_x, my_y, z_left),
            (my_x, my_y, z_right),
            (other_x, my_y, my_z),
        ):
            pl.semaphore_signal(
                barrier, inc=1,
                device_id=nbr_dev,
                device_id_type=pl.DeviceIdType.MESH,
            )
        pl.semaphore_wait(barrier, 5)

        my_b = N_X * my_z + my_x
        row0 = my_b * R

        def stage_dma(chunk_idx, h):
            return pltpu.make_async_copy(
                x_hbm.at[
                    pl.ds(row0, R),
                    pl.ds(chunk_idx * CHUNK + h * W, W),
                ],
                stage.at[h],
                copy_sems.at[h],
            )

        def rs_rdma(h, s):
            return pltpu.make_async_remote_copy(
                src_ref=rs_comm.at[h, s],
                dst_ref=rs_comm.at[h, s + 1],
                send_sem=rs_send.at[h, s],
                recv_sem=rs_recv.at[h, s],
                device_id=(my_x, y_right, my_z),
                device_id_type=pl.DeviceIdType.MESH,
            )

        def blk_rows(t, xi):
            zp = lax.rem(my_z + N_Z - t, N_Z)
            return (N_X * zp + xi) * R

        def ag_rdma(h, t):
            sl = (pl.ds(blk_rows(t, my_x), R), pl.ds(h * W, W))
            return pltpu.make_async_remote_copy(
                src_ref=out_ref.at[sl],
                dst_ref=out_ref.at[sl],
                send_sem=ag_send.at[h, t],
                recv_sem=ag_recv.at[h, t],
                device_id=(my_x, my_y, z_right),
                device_id_type=pl.DeviceIdType.MESH,
            )

        def x_pair(h, t):
            sl_mine = (pl.ds(blk_rows(t, my_x), R), pl.ds(h * W, W))
            sl_theirs = (pl.ds(blk_rows(t, other_x), R), pl.ds(h * W, W))
            send = pltpu.make_async_remote_copy(
                src_ref=out_ref.at[sl_mine],
                dst_ref=out_ref.at[sl_mine],
                send_sem=x_send.at[h, t],
                recv_sem=x_recv.at[h, t],
                device_id=(other_x, my_y, my_z),
                device_id_type=pl.DeviceIdType.MESH,
            )
            recv = pltpu.make_async_remote_copy(
                src_ref=out_ref.at[sl_theirs],
                dst_ref=out_ref.at[sl_theirs],
                send_sem=x_send.at[h, t],
                recv_sem=x_recv.at[h, t],
                device_id=(other_x, my_y, my_z),
                device_id_type=pl.DeviceIdType.MESH,
            )
            return send, recv

        x_pairs = [[x_pair(h, t) for t in range(N_Z)] for h in range(P)]

        c_seed = lax.rem(my_y + N_Y - 1, N_Y)

        def c_hop(s):
            return lax.rem(my_y + 2 * N_Y - 2 - s, N_Y)

        seed_dmas = []
        for h in range(P):
            d = stage_dma(c_seed, h)
            d.start()
            seed_dmas.append(d)
        for h in range(P):
            seed_dmas[h].wait()
            rs_comm[h, 0] = stage[h].astype(jnp.bfloat16)

        rs_d = [[rs_rdma(h, s) for s in range(N_Y - 1)] for h in range(P)]
        ag_d = [[ag_rdma(h, t) for t in range(N_Z - 1)] for h in range(P)]
        dmas = [None] * P

        def event(h, e):
            if e == 0:
                rs_d[h][0].start()
                d = stage_dma(c_hop(0), h)
                d.start()
                dmas[h] = d
            elif e <= 3:
                s = e - 1
                dmas[h].wait()
                rs_d[h][s].wait()
                contrib = stage[h].astype(jnp.bfloat16)
                if s < N_Y - 2:
                    rs_comm[h, s + 1] = rs_comm[h, s + 1] + contrib
                    rs_d[h][s + 1].start()
                    d = stage_dma(c_hop(s + 1), h)
                    d.start()
                    dmas[h] = d
                else:
                    out_ref[pl.ds(row0, R), pl.ds(h * W, W)] = (
                        rs_comm[h, s + 1] + contrib
                    )
                    x_pairs[h][0][0].start()
                    ag_d[h][0].start()
            else:
                t = e - 4
                ag_d[h][t].wait()
                x_pairs[h][t + 1][0].start()
                if t < N_Z - 2:
                    ag_d[h][t + 1].start()

        for k in range(N_EV + (P - 1) * OFF):
            for h in range(P):
                e = k - h * OFF
                if 0 <= e < N_EV:
                    event(h, e)

        for h in range(P):
            for t in range(N_Z):
                x_pairs[h][t][0].wait_send()
                x_pairs[h][t][1].wait_recv()

    return pl.pallas_call(
        body,
        out_shape=jax.ShapeDtypeStruct((M, CHUNK), jnp.bfloat16),
        in_specs=[pl.BlockSpec(memory_space=pl.ANY)],
        out_specs=pl.BlockSpec(memory_space=pltpu.VMEM),
        scratch_shapes=[
            pltpu.VMEM((P, N_Y, R, W), jnp.bfloat16),
            pltpu.VMEM((P, R, W), jnp.float32),
            pltpu.SemaphoreType.DMA((P,)),
            pltpu.SemaphoreType.DMA((P, N_Y - 1)),
            pltpu.SemaphoreType.DMA((P, N_Y - 1)),
            pltpu.SemaphoreType.DMA((P, N_Z - 1)),
            pltpu.SemaphoreType.DMA((P, N_Z - 1)),
            pltpu.SemaphoreType.DMA((P, N_Z)),
            pltpu.SemaphoreType.DMA((P, N_Z)),
        ],
        compiler_params=pltpu.CompilerParams(
            collective_id=0,
            vmem_limit_bytes=48 * 1024 * 1024,
        ),
    )(x2d)
